# baseline (speedup 1.0000x reference)
import functools

import jax
import jax.numpy as jnp
from jax import lax
from jax.experimental import pallas as pl
from jax.experimental.pallas import tpu as pltpu
from jax.experimental.pallas import tpu_sc as plsc

_K = 2
_BM_ROUTED = 512
_BM_SHARED = 512
_BM_ROUTER = 512


def _gelu_exact(h):
    return 0.5 * h * (1.0 + jax.lax.erf(h * 0.7071067811865476))


def _router_body(x_ref, w_ref, b_ref, eids_ref, gvals_ref):
    x = x_ref[...]
    w = w_ref[...]
    xh = x.astype(jnp.bfloat16)
    xl = (x - xh.astype(jnp.float32)).astype(jnp.bfloat16)
    wh = w.astype(jnp.bfloat16)
    wl = (w - wh.astype(jnp.float32)).astype(jnp.bfloat16)
    logits = (jnp.dot(xh, wh, preferred_element_type=jnp.float32)
              + jnp.dot(xh, wl, preferred_element_type=jnp.float32)
              + jnp.dot(xl, wh, preferred_element_type=jnp.float32)
              + b_ref[...])
    m = jnp.max(logits, axis=1, keepdims=True)
    ex = jnp.exp(logits - m)
    aff = ex / jnp.sum(ex, axis=1, keepdims=True)
    nr = aff.shape[1]
    iota = jax.lax.broadcasted_iota(jnp.int32, aff.shape, 1)
    m1 = jnp.max(aff, axis=1, keepdims=True)
    i1 = jnp.min(jnp.where(aff == m1, iota, nr), axis=1, keepdims=True)
    aff2 = jnp.where(iota == i1, -1.0, aff)
    m2 = jnp.max(aff2, axis=1, keepdims=True)
    i2 = jnp.min(jnp.where(aff2 == m2, iota, nr), axis=1, keepdims=True)
    eids_ref[...] = jnp.concatenate([i1, i2], axis=1)
    gvals_ref[...] = jnp.concatenate([m1, m2], axis=1)


def _shared_body(xb_ref, w1_ref, b1_ref, w2_ref, b2s_ref, out_ref):
    x = xb_ref[...]
    h = jnp.dot(x, w1_ref[...], preferred_element_type=jnp.float32) + b1_ref[...]
    h = _gelu_exact(h)
    y = jnp.dot(h.astype(jnp.bfloat16), w2_ref[...],
                preferred_element_type=jnp.float32)
    out_ref[...] = x.astype(jnp.float32) + b2s_ref[...] + y


def _grouped_body(eids_ref, x_ref, w1_ref, b1_ref, w2_ref, b2_ref, gate_ref,
                  out_ref):
    del eids_ref
    x = x_ref[...]
    h = jnp.dot(x, w1_ref[0], preferred_element_type=jnp.float32) + b1_ref[0]
    h = _gelu_exact(h)
    y = (jnp.dot(h.astype(jnp.bfloat16), w2_ref[0],
                 preferred_element_type=jnp.float32) + b2_ref[0])
    out_ref[...] = (y * gate_ref[...]).astype(jnp.bfloat16)


def _make_sc_gather(n_rows, n_src, H, dtype, n_chunks):
    info = plsc.get_sparse_core_info()
    nw = info.num_cores * info.num_subcores
    rows_per_w = n_rows // nw
    ch = rows_per_w // n_chunks
    mesh = plsc.VectorSubcoreMesh(core_axis_name="c", subcore_axis_name="s")

    @functools.partial(
        pl.kernel, mesh=mesh,
        out_type=jax.ShapeDtypeStruct((n_rows, H), dtype),
        scratch_types=[
            pltpu.VMEM((rows_per_w,), jnp.int32),
            pltpu.VMEM((ch, H), dtype),
            pltpu.VMEM((ch, H), dtype),
            pltpu.SemaphoreType.DMA,
            pltpu.SemaphoreType.DMA,
        ],
    )
    def k(src_hbm, idx_hbm, out_hbm, idx_v, rows_a, rows_b, sem_a, sem_b):
        wid = lax.axis_index("s") * info.num_cores + lax.axis_index("c")
        base = wid * rows_per_w
        pltpu.sync_copy(idx_hbm.at[pl.ds(base, rows_per_w)], idx_v)
        bufs = ((rows_a, sem_a), (rows_b, sem_b))
        copies = [None, None]
        for c in range(n_chunks):
            rows_v, sem = bufs[c % 2]
            if copies[c % 2] is not None:
                copies[c % 2].wait()
                pltpu.sync_copy(rows_v,
                                out_hbm.at[pl.ds(base + (c - 2) * ch, ch)])
            copies[c % 2] = pltpu.async_copy(
                src_hbm.at[idx_v.at[pl.ds(c * ch, ch)]], rows_v, sem)
        for c in range(n_chunks - 2, n_chunks):
            rows_v, sem = bufs[c % 2]
            copies[c % 2].wait()
            pltpu.sync_copy(rows_v, out_hbm.at[pl.ds(base + c * ch, ch)])

    return k


def kernel(x, shared_w1, shared_b1, shared_w2, shared_b2,
           routed_w1, routed_b1, routed_w2, routed_b2,
           router_w, router_b):
    B, S, H = x.shape
    NS, _, EI = shared_w1.shape
    NR = router_w.shape[1]
    T = B * S
    P = T * _K

    xf = x.reshape(T, H)
    xb = xf.astype(jnp.bfloat16)
    sw1 = shared_w1.astype(jnp.bfloat16)
    sw2 = shared_w2.astype(jnp.bfloat16)
    rw1 = routed_w1.astype(jnp.bfloat16)
    rw2 = routed_w2.astype(jnp.bfloat16)

    bm_r = min(_BM_ROUTER, T)
    eids, gvals = pl.pallas_call(
        _router_body,
        grid=(T // bm_r,),
        in_specs=[
            pl.BlockSpec((bm_r, H), lambda i: (i, 0)),
            pl.BlockSpec((H, NR), lambda i: (0, 0)),
            pl.BlockSpec((1, NR), lambda i: (0, 0)),
        ],
        out_specs=[
            pl.BlockSpec((bm_r, _K), lambda i: (i, 0)),
            pl.BlockSpec((bm_r, _K), lambda i: (i, 0)),
        ],
        out_shape=[
            jax.ShapeDtypeStruct((T, _K), jnp.int32),
            jax.ShapeDtypeStruct((T, _K), jnp.float32),
        ],
    )(xf, router_w, router_b.reshape(1, NR))

    bm = min(_BM_ROUTED, T)
    e_flat = eids.reshape(P)
    onehot = (e_flat[:, None] == jnp.arange(NR)[None, :]).astype(jnp.int32)
    cum = jnp.cumsum(onehot, axis=0)
    rank = jnp.take_along_axis(cum - onehot, e_flat[:, None], axis=1)[:, 0]
    counts = cum[-1]
    padded = ((counts + bm - 1) // bm) * bm
    offs = jnp.concatenate([jnp.zeros(1, jnp.int32),
                            jnp.cumsum(padded)[:-1].astype(jnp.int32)])
    dst = offs[e_flat] + rank
    NB = P // bm + NR
    Ppad = NB * bm
    n_chunks = 16
    info = plsc.get_sparse_core_info()
    align = info.num_cores * info.num_subcores * n_chunks * 8
    PG = ((Ppad + align - 1) // align) * align
    token_src = jnp.zeros(PG, jnp.int32).at[dst].set(
        jnp.arange(P, dtype=jnp.int32) // _K)
    gate_sorted = jnp.zeros((Ppad, 1), jnp.float32).at[dst, 0].set(
        gvals.reshape(P))
    block_eids = jnp.repeat(jnp.arange(NR, dtype=jnp.int32), padded // bm,
                            total_repeat_length=NB)
    xbp = lax.bitcast_convert_type(xb.reshape(T, H // 2, 2), jnp.int32)
    xs32 = _make_sc_gather(PG, T, H // 2, jnp.int32, n_chunks)(xbp, token_src)
    x_sorted = lax.bitcast_convert_type(xs32, jnp.bfloat16).reshape(PG, H)

    y_sorted = pl.pallas_call(
        _grouped_body,
        grid_spec=pltpu.PrefetchScalarGridSpec(
            num_scalar_prefetch=1,
            grid=(NB,),
            in_specs=[
                pl.BlockSpec((bm, H), lambda i, eids: (i, 0)),
                pl.BlockSpec((1, H, EI), lambda i, eids: (eids[i], 0, 0)),
                pl.BlockSpec((1, 1, EI), lambda i, eids: (eids[i], 0, 0)),
                pl.BlockSpec((1, EI, H), lambda i, eids: (eids[i], 0, 0)),
                pl.BlockSpec((1, 1, H), lambda i, eids: (eids[i], 0, 0)),
                pl.BlockSpec((bm, 1), lambda i, eids: (i, 0)),
            ],
            out_specs=pl.BlockSpec((bm, H), lambda i, eids: (i, 0)),
        ),
        out_shape=jax.ShapeDtypeStruct((Ppad, H), jnp.bfloat16),
    )(block_eids, x_sorted, rw1, routed_b1.reshape(NR, 1, EI), rw2,
      routed_b2.reshape(NR, 1, H), gate_sorted)

    bm_s = min(_BM_SHARED, T)
    w1cat = sw1.transpose(1, 0, 2).reshape(H, NS * EI)
    w2cat = sw2.reshape(NS * EI, H)
    b1cat = shared_b1.reshape(1, NS * EI)
    b2s = jnp.sum(shared_b2, axis=0).reshape(1, H)
    base = pl.pallas_call(
        _shared_body,
        grid=(T // bm_s,),
        in_specs=[
            pl.BlockSpec((bm_s, H), lambda i: (i, 0)),
            pl.BlockSpec((H, NS * EI), lambda i: (0, 0),
                         pipeline_mode=pl.Buffered(buffer_count=1)),
            pl.BlockSpec((1, NS * EI), lambda i: (0, 0)),
            pl.BlockSpec((NS * EI, H), lambda i: (0, 0),
                         pipeline_mode=pl.Buffered(buffer_count=1)),
            pl.BlockSpec((1, H), lambda i: (0, 0)),
        ],
        out_specs=pl.BlockSpec((bm_s, H), lambda i: (i, 0)),
        out_shape=jax.ShapeDtypeStruct((T, H), jnp.float32),
    )(xb, w1cat, b1cat, w2cat, b2s)

    pos = dst.reshape(T, _K)
    y1 = jax.lax.optimization_barrier(y_sorted[pos[:, 0]])
    y2 = jax.lax.optimization_barrier(y_sorted[pos[:, 1]])
    out = base + y1.astype(jnp.float32) + y2.astype(jnp.float32)
    return out.reshape(B, S, H)


# speedup vs baseline: 1.3161x; 1.3161x over previous
import functools

import jax
import jax.numpy as jnp
from jax import lax
from jax.experimental import pallas as pl
from jax.experimental.pallas import tpu as pltpu
from jax.experimental.pallas import tpu_sc as plsc

_K = 2
_BM_ROUTED = 512
_BM_SHARED = 512
_BM_ROUTER = 512


def _gelu_exact(h):
    return 0.5 * h * (1.0 + jax.lax.erf(h * 0.7071067811865476))


def _router_body(x_ref, w_ref, b_ref, eids_ref, gvals_ref):
    x = x_ref[...]
    w = w_ref[...]
    xh = x.astype(jnp.bfloat16)
    xl = (x - xh.astype(jnp.float32)).astype(jnp.bfloat16)
    wh = w.astype(jnp.bfloat16)
    wl = (w - wh.astype(jnp.float32)).astype(jnp.bfloat16)
    logits = (jnp.dot(xh, wh, preferred_element_type=jnp.float32)
              + jnp.dot(xh, wl, preferred_element_type=jnp.float32)
              + jnp.dot(xl, wh, preferred_element_type=jnp.float32)
              + b_ref[...])
    m = jnp.max(logits, axis=1, keepdims=True)
    ex = jnp.exp(logits - m)
    aff = ex / jnp.sum(ex, axis=1, keepdims=True)
    nr = aff.shape[1]
    iota = jax.lax.broadcasted_iota(jnp.int32, aff.shape, 1)
    m1 = jnp.max(aff, axis=1, keepdims=True)
    i1 = jnp.min(jnp.where(aff == m1, iota, nr), axis=1, keepdims=True)
    aff2 = jnp.where(iota == i1, -1.0, aff)
    m2 = jnp.max(aff2, axis=1, keepdims=True)
    i2 = jnp.min(jnp.where(aff2 == m2, iota, nr), axis=1, keepdims=True)
    eids_ref[...] = jnp.concatenate([i1, i2], axis=1)
    gvals_ref[...] = jnp.concatenate([m1, m2], axis=1)


def _shared_body(xb_ref, w1_ref, b1_ref, w2_ref, b2s_ref, out_ref):
    x = xb_ref[...]
    h = jnp.dot(x, w1_ref[...], preferred_element_type=jnp.float32) + b1_ref[...]
    h = _gelu_exact(h)
    y = jnp.dot(h.astype(jnp.bfloat16), w2_ref[...],
                preferred_element_type=jnp.float32)
    out_ref[...] = x.astype(jnp.float32) + b2s_ref[...] + y


def _grouped_body(eids_ref, x_ref, w1_ref, b1_ref, w2_ref, b2_ref, gate_ref,
                  out_ref):
    del eids_ref
    x = x_ref[...]
    h = jnp.dot(x, w1_ref[0], preferred_element_type=jnp.float32) + b1_ref[0]
    h = _gelu_exact(h)
    y = (jnp.dot(h.astype(jnp.bfloat16), w2_ref[0],
                 preferred_element_type=jnp.float32) + b2_ref[0])
    out_ref[...] = (y * gate_ref[...]).astype(jnp.bfloat16)


def _make_sc_gather(n_rows, n_src, H, dtype, n_chunks):
    info = plsc.get_sparse_core_info()
    nw = info.num_cores * info.num_subcores
    rows_per_w = n_rows // nw
    ch = rows_per_w // n_chunks
    mesh = plsc.VectorSubcoreMesh(core_axis_name="c", subcore_axis_name="s")

    @functools.partial(
        pl.kernel, mesh=mesh,
        out_type=jax.ShapeDtypeStruct((n_rows, H), dtype),
        scratch_types=[
            pltpu.VMEM((rows_per_w,), jnp.int32),
            pltpu.VMEM((ch, H), dtype),
            pltpu.VMEM((ch, H), dtype),
            pltpu.SemaphoreType.DMA,
            pltpu.SemaphoreType.DMA,
        ],
    )
    def k(src_hbm, idx_hbm, out_hbm, idx_v, rows_a, rows_b, sem_a, sem_b):
        wid = lax.axis_index("s") * info.num_cores + lax.axis_index("c")
        base = wid * rows_per_w
        pltpu.sync_copy(idx_hbm.at[pl.ds(base, rows_per_w)], idx_v)
        bufs = ((rows_a, sem_a), (rows_b, sem_b))
        copies = [None, None]
        for c in range(n_chunks):
            rows_v, sem = bufs[c % 2]
            if copies[c % 2] is not None:
                copies[c % 2].wait()
                pltpu.sync_copy(rows_v,
                                out_hbm.at[pl.ds(base + (c - 2) * ch, ch)])
            copies[c % 2] = pltpu.async_copy(
                src_hbm.at[idx_v.at[pl.ds(c * ch, ch)]], rows_v, sem)
        for c in range(n_chunks - 2, n_chunks):
            rows_v, sem = bufs[c % 2]
            copies[c % 2].wait()
            pltpu.sync_copy(rows_v, out_hbm.at[pl.ds(base + c * ch, ch)])

    return k


def kernel(x, shared_w1, shared_b1, shared_w2, shared_b2,
           routed_w1, routed_b1, routed_w2, routed_b2,
           router_w, router_b):
    B, S, H = x.shape
    NS, _, EI = shared_w1.shape
    NR = router_w.shape[1]
    T = B * S
    P = T * _K

    xf = x.reshape(T, H)
    xb = xf.astype(jnp.bfloat16)
    sw1 = shared_w1.astype(jnp.bfloat16)
    sw2 = shared_w2.astype(jnp.bfloat16)
    rw1 = routed_w1.astype(jnp.bfloat16)
    rw2 = routed_w2.astype(jnp.bfloat16)

    bm_r = min(_BM_ROUTER, T)
    eids, gvals = pl.pallas_call(
        _router_body,
        grid=(T // bm_r,),
        in_specs=[
            pl.BlockSpec((bm_r, H), lambda i: (i, 0)),
            pl.BlockSpec((H, NR), lambda i: (0, 0)),
            pl.BlockSpec((1, NR), lambda i: (0, 0)),
        ],
        out_specs=[
            pl.BlockSpec((bm_r, _K), lambda i: (i, 0)),
            pl.BlockSpec((bm_r, _K), lambda i: (i, 0)),
        ],
        out_shape=[
            jax.ShapeDtypeStruct((T, _K), jnp.int32),
            jax.ShapeDtypeStruct((T, _K), jnp.float32),
        ],
    )(xf, router_w, router_b.reshape(1, NR))

    bm = min(_BM_ROUTED, T)
    e_flat = eids.reshape(P)
    onehot = (e_flat[:, None] == jnp.arange(NR)[None, :]).astype(jnp.int32)
    cum = jnp.cumsum(onehot, axis=0)
    rank = jnp.take_along_axis(cum - onehot, e_flat[:, None], axis=1)[:, 0]
    counts = cum[-1]
    padded = ((counts + bm - 1) // bm) * bm
    offs = jnp.concatenate([jnp.zeros(1, jnp.int32),
                            jnp.cumsum(padded)[:-1].astype(jnp.int32)])
    dst = offs[e_flat] + rank
    NB = P // bm + NR
    Ppad = NB * bm
    n_chunks = 16
    info = plsc.get_sparse_core_info()
    align = info.num_cores * info.num_subcores * n_chunks * 8
    PG = ((Ppad + align - 1) // align) * align
    token_src = jnp.zeros(PG, jnp.int32).at[dst].set(
        jnp.arange(P, dtype=jnp.int32) // _K)
    gate_sorted = jnp.zeros((Ppad, 1), jnp.float32).at[dst, 0].set(
        gvals.reshape(P))
    block_eids = jnp.repeat(jnp.arange(NR, dtype=jnp.int32), padded // bm,
                            total_repeat_length=NB)
    halves = [jnp.take(xb, token_src[off:off + PG // 2], axis=0)
              for off in (0, PG // 2)]
    x_sorted = jnp.concatenate(halves, axis=0)

    y_sorted = pl.pallas_call(
        _grouped_body,
        grid_spec=pltpu.PrefetchScalarGridSpec(
            num_scalar_prefetch=1,
            grid=(NB,),
            in_specs=[
                pl.BlockSpec((bm, H), lambda i, eids: (i, 0)),
                pl.BlockSpec((1, H, EI), lambda i, eids: (eids[i], 0, 0)),
                pl.BlockSpec((1, 1, EI), lambda i, eids: (eids[i], 0, 0)),
                pl.BlockSpec((1, EI, H), lambda i, eids: (eids[i], 0, 0)),
                pl.BlockSpec((1, 1, H), lambda i, eids: (eids[i], 0, 0)),
                pl.BlockSpec((bm, 1), lambda i, eids: (i, 0)),
            ],
            out_specs=pl.BlockSpec((bm, H), lambda i, eids: (i, 0)),
        ),
        out_shape=jax.ShapeDtypeStruct((Ppad, H), jnp.bfloat16),
    )(block_eids, x_sorted, rw1, routed_b1.reshape(NR, 1, EI), rw2,
      routed_b2.reshape(NR, 1, H), gate_sorted)

    bm_s = min(_BM_SHARED, T)
    w1cat = sw1.transpose(1, 0, 2).reshape(H, NS * EI)
    w2cat = sw2.reshape(NS * EI, H)
    b1cat = shared_b1.reshape(1, NS * EI)
    b2s = jnp.sum(shared_b2, axis=0).reshape(1, H)
    base = pl.pallas_call(
        _shared_body,
        grid=(T // bm_s,),
        in_specs=[
            pl.BlockSpec((bm_s, H), lambda i: (i, 0)),
            pl.BlockSpec((H, NS * EI), lambda i: (0, 0),
                         pipeline_mode=pl.Buffered(buffer_count=1)),
            pl.BlockSpec((1, NS * EI), lambda i: (0, 0)),
            pl.BlockSpec((NS * EI, H), lambda i: (0, 0),
                         pipeline_mode=pl.Buffered(buffer_count=1)),
            pl.BlockSpec((1, H), lambda i: (0, 0)),
        ],
        out_specs=pl.BlockSpec((bm_s, H), lambda i: (i, 0)),
        out_shape=jax.ShapeDtypeStruct((T, H), jnp.float32),
    )(xb, w1cat, b1cat, w2cat, b2s)

    pos = dst.reshape(T, _K)
    y1 = jax.lax.optimization_barrier(y_sorted[pos[:, 0]])
    y2 = jax.lax.optimization_barrier(y_sorted[pos[:, 1]])
    out = base + y1.astype(jnp.float32) + y2.astype(jnp.float32)
    return out.reshape(B, S, H)


# speedup vs baseline: 1.9217x; 1.4602x over previous
import functools

import jax
import jax.numpy as jnp
from jax import lax
from jax.experimental import pallas as pl
from jax.experimental.pallas import tpu as pltpu
from jax.experimental.pallas import tpu_sc as plsc

_K = 2
_BM_ROUTED = 256
_BM_SHARED = 512
_BM_ROUTER = 512


def _gelu_exact(h):
    return 0.5 * h * (1.0 + jax.lax.erf(h * 0.7071067811865476))


def _router_body(x_ref, w_ref, b_ref, eids_ref, gvals_ref):
    x = x_ref[...]
    w = w_ref[...]
    xh = x.astype(jnp.bfloat16)
    xl = (x - xh.astype(jnp.float32)).astype(jnp.bfloat16)
    wh = w.astype(jnp.bfloat16)
    wl = (w - wh.astype(jnp.float32)).astype(jnp.bfloat16)
    logits = (jnp.dot(xh, wh, preferred_element_type=jnp.float32)
              + jnp.dot(xh, wl, preferred_element_type=jnp.float32)
              + jnp.dot(xl, wh, preferred_element_type=jnp.float32)
              + b_ref[...])
    m = jnp.max(logits, axis=1, keepdims=True)
    ex = jnp.exp(logits - m)
    aff = ex / jnp.sum(ex, axis=1, keepdims=True)
    nr = aff.shape[1]
    iota = jax.lax.broadcasted_iota(jnp.int32, aff.shape, 1)
    m1 = jnp.max(aff, axis=1, keepdims=True)
    i1 = jnp.min(jnp.where(aff == m1, iota, nr), axis=1, keepdims=True)
    aff2 = jnp.where(iota == i1, -1.0, aff)
    m2 = jnp.max(aff2, axis=1, keepdims=True)
    i2 = jnp.min(jnp.where(aff2 == m2, iota, nr), axis=1, keepdims=True)
    eids_ref[...] = jnp.concatenate([i1, i2], axis=1)
    gvals_ref[...] = jnp.concatenate([m1, m2], axis=1)


def _shared_body(xb_ref, w1_ref, b1_ref, w2_ref, b2s_ref, out_ref):
    x = xb_ref[...]
    h = jnp.dot(x, w1_ref[...], preferred_element_type=jnp.float32) + b1_ref[...]
    h = _gelu_exact(h)
    y = jnp.dot(h.astype(jnp.bfloat16), w2_ref[...],
                preferred_element_type=jnp.float32)
    out_ref[...] = x.astype(jnp.float32) + b2s_ref[...] + y


def _grouped_body(eids_ref, x_ref, w1_ref, b1_ref, w2_ref, b2_ref, gate_ref,
                  out_ref):
    del eids_ref
    x = x_ref[...]
    h = jnp.dot(x, w1_ref[0], preferred_element_type=jnp.float32) + b1_ref[0]
    h = _gelu_exact(h)
    y = (jnp.dot(h.astype(jnp.bfloat16), w2_ref[0],
                 preferred_element_type=jnp.float32) + b2_ref[0])
    out_ref[...] = (y * gate_ref[...]).astype(jnp.bfloat16)


def kernel(x, shared_w1, shared_b1, shared_w2, shared_b2,
           routed_w1, routed_b1, routed_w2, routed_b2,
           router_w, router_b):
    B, S, H = x.shape
    NS, _, EI = shared_w1.shape
    NR = router_w.shape[1]
    T = B * S
    P = T * _K

    xf = x.reshape(T, H)
    xb = xf.astype(jnp.bfloat16)
    sw1 = shared_w1.astype(jnp.bfloat16)
    sw2 = shared_w2.astype(jnp.bfloat16)
    rw1 = routed_w1.astype(jnp.bfloat16)
    rw2 = routed_w2.astype(jnp.bfloat16)

    bm_r = min(_BM_ROUTER, T)
    eids, gvals = pl.pallas_call(
        _router_body,
        grid=(T // bm_r,),
        in_specs=[
            pl.BlockSpec((bm_r, H), lambda i: (i, 0)),
            pl.BlockSpec((H, NR), lambda i: (0, 0)),
            pl.BlockSpec((1, NR), lambda i: (0, 0)),
        ],
        out_specs=[
            pl.BlockSpec((bm_r, _K), lambda i: (i, 0)),
            pl.BlockSpec((bm_r, _K), lambda i: (i, 0)),
        ],
        out_shape=[
            jax.ShapeDtypeStruct((T, _K), jnp.int32),
            jax.ShapeDtypeStruct((T, _K), jnp.float32),
        ],
    )(xf, router_w, router_b.reshape(1, NR))

    bm = min(_BM_ROUTED, T)
    e_flat = eids.reshape(P)
    onehot = (e_flat[:, None] == jnp.arange(NR)[None, :]).astype(jnp.int32)
    cum = jnp.cumsum(onehot, axis=0)
    rank = jnp.take_along_axis(cum - onehot, e_flat[:, None], axis=1)[:, 0]
    counts = cum[-1]
    padded = ((counts + bm - 1) // bm) * bm
    offs = jnp.concatenate([jnp.zeros(1, jnp.int32),
                            jnp.cumsum(padded)[:-1].astype(jnp.int32)])
    dst = offs[e_flat] + rank
    NB = P // bm + NR
    Ppad = NB * bm
    pairs = jnp.stack([(jnp.arange(P, dtype=jnp.int32) // _K)
                       .astype(jnp.float32), gvals.reshape(P)], axis=1)
    scat = jnp.zeros((Ppad, 2), jnp.float32).at[dst].set(pairs)
    token_src = scat[:, 0].astype(jnp.int32)
    gate_sorted = scat[:, 1:2]
    block_eids = jnp.repeat(jnp.arange(NR, dtype=jnp.int32), padded // bm,
                            total_repeat_length=NB)
    x_sorted = jax.lax.optimization_barrier(xb)[token_src]

    y_sorted = pl.pallas_call(
        _grouped_body,
        grid_spec=pltpu.PrefetchScalarGridSpec(
            num_scalar_prefetch=1,
            grid=(NB,),
            in_specs=[
                pl.BlockSpec((bm, H), lambda i, eids: (i, 0)),
                pl.BlockSpec((1, H, EI), lambda i, eids: (eids[i], 0, 0)),
                pl.BlockSpec((1, 1, EI), lambda i, eids: (eids[i], 0, 0)),
                pl.BlockSpec((1, EI, H), lambda i, eids: (eids[i], 0, 0)),
                pl.BlockSpec((1, 1, H), lambda i, eids: (eids[i], 0, 0)),
                pl.BlockSpec((bm, 1), lambda i, eids: (i, 0)),
            ],
            out_specs=pl.BlockSpec((bm, H), lambda i, eids: (i, 0)),
        ),
        out_shape=jax.ShapeDtypeStruct((Ppad, H), jnp.bfloat16),
    )(block_eids, x_sorted, rw1, routed_b1.reshape(NR, 1, EI), rw2,
      routed_b2.reshape(NR, 1, H), gate_sorted)

    bm_s = min(_BM_SHARED, T)
    w1cat = sw1.transpose(1, 0, 2).reshape(H, NS * EI)
    w2cat = sw2.reshape(NS * EI, H)
    b1cat = shared_b1.reshape(1, NS * EI)
    b2s = jnp.sum(shared_b2, axis=0).reshape(1, H)
    base = pl.pallas_call(
        _shared_body,
        grid=(T // bm_s,),
        in_specs=[
            pl.BlockSpec((bm_s, H), lambda i: (i, 0)),
            pl.BlockSpec((H, NS * EI), lambda i: (0, 0),
                         pipeline_mode=pl.Buffered(buffer_count=1)),
            pl.BlockSpec((1, NS * EI), lambda i: (0, 0)),
            pl.BlockSpec((NS * EI, H), lambda i: (0, 0),
                         pipeline_mode=pl.Buffered(buffer_count=1)),
            pl.BlockSpec((1, H), lambda i: (0, 0)),
        ],
        out_specs=pl.BlockSpec((bm_s, H), lambda i: (i, 0)),
        out_shape=jax.ShapeDtypeStruct((T, H), jnp.float32),
    )(xb, w1cat, b1cat, w2cat, b2s)

    pos = dst.reshape(T, _K)
    y1 = jax.lax.optimization_barrier(y_sorted[pos[:, 0]])
    y2 = jax.lax.optimization_barrier(y_sorted[pos[:, 1]])
    out = base + y1.astype(jnp.float32) + y2.astype(jnp.float32)
    return out.reshape(B, S, H)


# speedup vs baseline: 1.9346x; 1.0067x over previous
import functools

import jax
import jax.numpy as jnp
from jax import lax
from jax.experimental import pallas as pl
from jax.experimental.pallas import tpu as pltpu
from jax.experimental.pallas import tpu_sc as plsc

_K = 2
_BM_ROUTED = 256
_BM_SHARED = 512
_BM_ROUTER = 512


def _gelu_exact(h):
    return 0.5 * h * (1.0 + jax.lax.erf(h * 0.7071067811865476))


def _router_body(x_ref, w_ref, b_ref, eids_ref, gvals_ref):
    x = x_ref[...]
    w = w_ref[...]
    xh = x.astype(jnp.bfloat16)
    xl = (x - xh.astype(jnp.float32)).astype(jnp.bfloat16)
    wh = w.astype(jnp.bfloat16)
    wl = (w - wh.astype(jnp.float32)).astype(jnp.bfloat16)
    logits = (jnp.dot(xh, wh, preferred_element_type=jnp.float32)
              + jnp.dot(xh, wl, preferred_element_type=jnp.float32)
              + jnp.dot(xl, wh, preferred_element_type=jnp.float32)
              + b_ref[...])
    m = jnp.max(logits, axis=1, keepdims=True)
    ex = jnp.exp(logits - m)
    aff = ex / jnp.sum(ex, axis=1, keepdims=True)
    nr = aff.shape[1]
    iota = jax.lax.broadcasted_iota(jnp.int32, aff.shape, 1)
    m1 = jnp.max(aff, axis=1, keepdims=True)
    i1 = jnp.min(jnp.where(aff == m1, iota, nr), axis=1, keepdims=True)
    aff2 = jnp.where(iota == i1, -1.0, aff)
    m2 = jnp.max(aff2, axis=1, keepdims=True)
    i2 = jnp.min(jnp.where(aff2 == m2, iota, nr), axis=1, keepdims=True)
    eids_ref[...] = jnp.concatenate([i1, i2], axis=1)
    gvals_ref[...] = jnp.concatenate([m1, m2], axis=1)


def _shared_body(xb_ref, w1_ref, b1_ref, w2_ref, b2s_ref, y1_ref, y2_ref,
                 out_ref):
    x = xb_ref[...]
    h = jnp.dot(x, w1_ref[...], preferred_element_type=jnp.float32) + b1_ref[...]
    h = _gelu_exact(h)
    y = jnp.dot(h.astype(jnp.bfloat16), w2_ref[...],
                preferred_element_type=jnp.float32)
    out_ref[...] = (x.astype(jnp.float32) + b2s_ref[...] + y
                    + y1_ref[...].astype(jnp.float32)
                    + y2_ref[...].astype(jnp.float32))


def _grouped_body(eids_ref, x_ref, w1_ref, b1_ref, w2_ref, b2_ref, gate_ref,
                  out_ref):
    del eids_ref
    x = x_ref[...]
    h = jnp.dot(x, w1_ref[0], preferred_element_type=jnp.float32) + b1_ref[0]
    h = _gelu_exact(h)
    y = (jnp.dot(h.astype(jnp.bfloat16), w2_ref[0],
                 preferred_element_type=jnp.float32) + b2_ref[0])
    out_ref[...] = (y * gate_ref[...]).astype(jnp.bfloat16)


def kernel(x, shared_w1, shared_b1, shared_w2, shared_b2,
           routed_w1, routed_b1, routed_w2, routed_b2,
           router_w, router_b):
    B, S, H = x.shape
    NS, _, EI = shared_w1.shape
    NR = router_w.shape[1]
    T = B * S
    P = T * _K

    xf = x.reshape(T, H)
    xb = xf.astype(jnp.bfloat16)
    sw1 = shared_w1.astype(jnp.bfloat16)
    sw2 = shared_w2.astype(jnp.bfloat16)
    rw1 = routed_w1.astype(jnp.bfloat16)
    rw2 = routed_w2.astype(jnp.bfloat16)

    bm_r = min(_BM_ROUTER, T)
    eids, gvals = pl.pallas_call(
        _router_body,
        grid=(T // bm_r,),
        in_specs=[
            pl.BlockSpec((bm_r, H), lambda i: (i, 0)),
            pl.BlockSpec((H, NR), lambda i: (0, 0)),
            pl.BlockSpec((1, NR), lambda i: (0, 0)),
        ],
        out_specs=[
            pl.BlockSpec((bm_r, _K), lambda i: (i, 0)),
            pl.BlockSpec((bm_r, _K), lambda i: (i, 0)),
        ],
        out_shape=[
            jax.ShapeDtypeStruct((T, _K), jnp.int32),
            jax.ShapeDtypeStruct((T, _K), jnp.float32),
        ],
    )(xf, router_w, router_b.reshape(1, NR))

    bm = min(_BM_ROUTED, T)
    e_flat = eids.reshape(P)
    onehot = (e_flat[:, None] == jnp.arange(NR)[None, :]).astype(jnp.int32)
    cum = jnp.cumsum(onehot, axis=0)
    rank = jnp.take_along_axis(cum - onehot, e_flat[:, None], axis=1)[:, 0]
    counts = cum[-1]
    padded = ((counts + bm - 1) // bm) * bm
    offs = jnp.concatenate([jnp.zeros(1, jnp.int32),
                            jnp.cumsum(padded)[:-1].astype(jnp.int32)])
    dst = offs[e_flat] + rank
    NB = P // bm + NR
    Ppad = NB * bm
    pairs = jnp.stack([(jnp.arange(P, dtype=jnp.int32) // _K)
                       .astype(jnp.float32), gvals.reshape(P)], axis=1)
    scat = jnp.zeros((Ppad, 2), jnp.float32).at[dst].set(pairs)
    token_src = scat[:, 0].astype(jnp.int32)
    gate_sorted = scat[:, 1:2]
    block_eids = jnp.repeat(jnp.arange(NR, dtype=jnp.int32), padded // bm,
                            total_repeat_length=NB)
    x_sorted = jax.lax.optimization_barrier(xb)[token_src]

    y_sorted = pl.pallas_call(
        _grouped_body,
        grid_spec=pltpu.PrefetchScalarGridSpec(
            num_scalar_prefetch=1,
            grid=(NB,),
            in_specs=[
                pl.BlockSpec((bm, H), lambda i, eids: (i, 0)),
                pl.BlockSpec((1, H, EI), lambda i, eids: (eids[i], 0, 0)),
                pl.BlockSpec((1, 1, EI), lambda i, eids: (eids[i], 0, 0)),
                pl.BlockSpec((1, EI, H), lambda i, eids: (eids[i], 0, 0)),
                pl.BlockSpec((1, 1, H), lambda i, eids: (eids[i], 0, 0)),
                pl.BlockSpec((bm, 1), lambda i, eids: (i, 0)),
            ],
            out_specs=pl.BlockSpec((bm, H), lambda i, eids: (i, 0)),
        ),
        out_shape=jax.ShapeDtypeStruct((Ppad, H), jnp.bfloat16),
    )(block_eids, x_sorted, rw1, routed_b1.reshape(NR, 1, EI), rw2,
      routed_b2.reshape(NR, 1, H), gate_sorted)

    pos = dst.reshape(T, _K)
    y1 = jax.lax.optimization_barrier(y_sorted[pos[:, 0]])
    y2 = jax.lax.optimization_barrier(y_sorted[pos[:, 1]])

    bm_s = min(_BM_SHARED, T)
    w1cat = sw1.transpose(1, 0, 2).reshape(H, NS * EI)
    w2cat = sw2.reshape(NS * EI, H)
    b1cat = shared_b1.reshape(1, NS * EI)
    b2s = jnp.sum(shared_b2, axis=0).reshape(1, H)
    out = pl.pallas_call(
        _shared_body,
        grid=(T // bm_s,),
        in_specs=[
            pl.BlockSpec((bm_s, H), lambda i: (i, 0)),
            pl.BlockSpec((H, NS * EI), lambda i: (0, 0),
                         pipeline_mode=pl.Buffered(buffer_count=1)),
            pl.BlockSpec((1, NS * EI), lambda i: (0, 0)),
            pl.BlockSpec((NS * EI, H), lambda i: (0, 0),
                         pipeline_mode=pl.Buffered(buffer_count=1)),
            pl.BlockSpec((1, H), lambda i: (0, 0)),
            pl.BlockSpec((bm_s, H), lambda i: (i, 0)),
            pl.BlockSpec((bm_s, H), lambda i: (i, 0)),
        ],
        out_specs=pl.BlockSpec((bm_s, H), lambda i: (i, 0)),
        out_shape=jax.ShapeDtypeStruct((T, H), jnp.float32),
    )(xb, w1cat, b1cat, w2cat, b2s, y1, y2)
    return out.reshape(B, S, H)
